# baseline (speedup 1.0000x reference)
import functools

import jax
import jax.numpy as jnp
from jax import lax
from jax.experimental import pallas as pl
from jax.experimental.pallas import tpu as pltpu
from jax.experimental.pallas import tpu_sc as plsc

DICT_LEN = 1000000
LATENT_SIZE = 32

S = 200
HA = 128
HB = S - HA
K = 8

NC = 2
NS = 16
NW = NC * NS


def _embed_kernel(n_iters, idx_hbm, table_hbm, out_hbm, idx_a, idx_b, rows_v,
                  sem0, sem1):
    wid = lax.axis_index("s") * NC + lax.axis_index("c")
    base = wid * n_iters * K
    sems = (sem0, sem1)

    def stage_and_fire(s, g):
        b0 = base + g * K
        pltpu.sync_copy(idx_hbm.at[pl.ds(b0, K), pl.ds(0, HA)], idx_a.at[s])
        pltpu.sync_copy(idx_hbm.at[pl.ds(b0, K), pl.ds(HA, HB)], idx_b.at[s])
        for j in range(K):
            pltpu.async_copy(
                table_hbm.at[idx_a.at[s].at[j]],
                rows_v.at[s].at[j, pl.ds(0, HA)],
                sems[s],
            )
            pltpu.async_copy(
                table_hbm.at[idx_b.at[s].at[j]],
                rows_v.at[s].at[j, pl.ds(HA, HB)],
                sems[s],
            )

    def drain_and_write(s, g):
        pltpu.make_async_copy(
            out_hbm.at[pl.ds(0, K)], rows_v.at[s], sems[s]
        ).wait()
        pltpu.sync_copy(rows_v.at[s], out_hbm.at[pl.ds(base + g * K, K)])

    stage_and_fire(0, 0)

    def body(p, carry):
        g = 2 * p
        stage_and_fire(1, g + 1)
        drain_and_write(0, g)
        stage_and_fire(0, g + 2)
        drain_and_write(1, g + 1)
        return carry

    lax.fori_loop(0, n_iters // 2 - 1, body, 0)

    g = n_iters - 2
    stage_and_fire(1, g + 1)
    drain_and_write(0, g)
    drain_and_write(1, g + 1)


def kernel(x, latent_tdirs):
    b, s = x.shape[0], x.shape[1]
    assert s == S and b % (NW * K) == 0
    n_iters = b // (NW * K)

    mesh = plsc.VectorSubcoreMesh(core_axis_name="c", subcore_axis_name="s")
    run = functools.partial(
        pl.kernel,
        mesh=mesh,
        compiler_params=pltpu.CompilerParams(use_tc_tiling_on_sc=False),
        out_type=jax.ShapeDtypeStruct((b, S, LATENT_SIZE), jnp.float32),
        scratch_types=[
            pltpu.VMEM((2, K, HA), jnp.int32),
            pltpu.VMEM((2, K, HB), jnp.int32),
            pltpu.VMEM((2, K, S, LATENT_SIZE), jnp.float32),
            pltpu.SemaphoreType.DMA,
            pltpu.SemaphoreType.DMA,
        ],
    )(functools.partial(_embed_kernel, n_iters))

    return run(jnp.squeeze(x, -1).astype(jnp.int32), latent_tdirs)
